# baseline (speedup 1.0000x reference)
import functools

import jax
import jax.numpy as jnp
from jax.experimental import pallas as pl
from jax.experimental.pallas import tpu as pltpu

N = 10000
E = 160000
IN = 128
HID = 64
H1 = 8
OUT = 256
NC = 60

_BN = 1000


def _dense1_body(h_ref, w_ref, a_ref, feat_ref, elr_ref):
    h = h_ref[...]
    feat = jnp.dot(h, w_ref[...], preferred_element_type=jnp.float32)
    feat_ref[...] = feat
    elr_ref[...] = jnp.dot(feat, a_ref[...], preferred_element_type=jnp.float32)


def _dense1(h, W, A):
    K = h.shape[1]
    F = W.shape[1]
    twoH = A.shape[1]
    grid = (N // _BN,)
    return pl.pallas_call(
        _dense1_body,
        grid=grid,
        in_specs=[
            pl.BlockSpec((_BN, K), lambda i: (i, 0)),
            pl.BlockSpec((K, F), lambda i: (0, 0)),
            pl.BlockSpec((F, twoH), lambda i: (0, 0)),
        ],
        out_specs=[
            pl.BlockSpec((_BN, F), lambda i: (i, 0)),
            pl.BlockSpec((_BN, twoH), lambda i: (i, 0)),
        ],
        out_shape=[
            jax.ShapeDtypeStruct((N, F), jnp.float32),
            jax.ShapeDtypeStruct((N, twoH), jnp.float32),
        ],
    )(h, W, A)


def kernel(node_2d_features, feat_2d, edge_index, W1, al1, ar1, b1, W2, al2,
           ar2, b2, Wc, bc):
    src, dst = edge_index[0], edge_index[1]

    h0 = jnp.concatenate([node_2d_features, feat_2d], axis=1)
    eye = jnp.eye(H1, dtype=jnp.float32)
    A1l = (al1.reshape(H1, 1, HID) * eye[:, :, None]).transpose(1, 2, 0).reshape(H1 * HID, H1)
    A1r = (ar1.reshape(H1, 1, HID) * eye[:, :, None]).transpose(1, 2, 0).reshape(H1 * HID, H1)
    A1 = jnp.concatenate([A1l, A1r], axis=1)
    feat1, elr1 = _dense1(h0, W1, A1)
    el1, er1 = elr1[:, :H1], elr1[:, H1:]

    h1 = _edge_layer_xla(feat1, el1, er1, src, dst, H1, HID, b1, True)

    A2 = jnp.concatenate([al2.reshape(OUT, 1), ar2.reshape(OUT, 1)], axis=1)
    feat2, elr2 = _dense1(h1, W2, A2)
    el2, er2 = elr2[:, :1], elr2[:, 1:]

    h2 = _edge_layer_xla(feat2, el2, er2, src, dst, 1, OUT, b2, False)

    y = h2.mean(axis=0, keepdims=True)
    label = y @ Wc + bc
    return (h2, label)


def _edge_layer_xla(feat_flat, el, er, src, dst, H, D, b, apply_elu):
    feat = feat_flat.reshape(N, H, D)
    e = jax.nn.leaky_relu(el[src] + er[dst], negative_slope=0.2)
    emax = jax.ops.segment_max(e, dst, num_segments=N)
    emax = jnp.where(jnp.isfinite(emax), emax, 0.0)
    ee = jnp.exp(e - emax[dst])
    denom = jax.ops.segment_sum(ee, dst, num_segments=N)
    alpha = ee / jnp.maximum(denom[dst], 1e-9)
    msg = feat[src] * alpha[..., None]
    rst = jax.ops.segment_sum(msg, dst, num_segments=N)
    rst = rst + b.reshape(1, H, D)
    if apply_elu:
        rst = jax.nn.elu(rst)
    return rst.reshape(N, H * D)


# speedup vs baseline: 3.3079x; 3.3079x over previous
import functools

import jax
import jax.numpy as jnp
from jax import lax
from jax.experimental import pallas as pl
from jax.experimental.pallas import tpu as pltpu
from jax.experimental.pallas import tpu_sc as plsc

N = 10000
E = 160000
IN = 128
HID = 64
H1 = 8
OUT = 256
NC = 60

_BN = 1000
_NP = 10240
_B = 128
_NT = 16
_CW = 128


def _dense1_body(h_ref, w_ref, a_ref, feat_ref, elr_ref):
    h = h_ref[...]
    feat = jnp.dot(h, w_ref[...], preferred_element_type=jnp.float32)
    feat_ref[...] = feat
    elr_ref[...] = jnp.dot(feat, a_ref[...], preferred_element_type=jnp.float32)


def _dense1(h, W, A):
    K = h.shape[1]
    F = W.shape[1]
    twoH = A.shape[1]
    grid = (N // _BN,)
    return pl.pallas_call(
        _dense1_body,
        grid=grid,
        in_specs=[
            pl.BlockSpec((_BN, K), lambda i: (i, 0)),
            pl.BlockSpec((K, F), lambda i: (0, 0)),
            pl.BlockSpec((F, twoH), lambda i: (0, 0)),
        ],
        out_specs=[
            pl.BlockSpec((_BN, F), lambda i: (i, 0)),
            pl.BlockSpec((_BN, twoH), lambda i: (i, 0)),
        ],
        out_shape=[
            jax.ShapeDtypeStruct((N, F), jnp.float32),
            jax.ShapeDtypeStruct((N, twoH), jnp.float32),
        ],
    )(h, W, A)


_DNUMS = lax.GatherDimensionNumbers(
    offset_dims=(), collapsed_slice_dims=(0,), start_index_map=(0,))


def _bcast(vec, lane):
    idx = jnp.full((16,), lane, jnp.int32)
    return lax.gather(vec, idx[:, None], _DNUMS, (1,),
                      mode=lax.GatherScatterMode.PROMISE_IN_BOUNDS)


def _agg_body(nchunks, H, ept, feat_ref, src_ref, dst_ref, alpha_ref, out_ref,
              src_v, dst_v, alpha_v, idx_v, rows_v, accum, sem):
    co = lax.axis_index("c")
    sid = lax.axis_index("s")
    cpc = nchunks // 2
    seg = _CW // (HID if H > 1 else _CW)
    scols = _CW // seg
    nb = ept // _B
    rslice = _NP // _NT

    for cc in range(cpc):
        c = co * cpc + cc

        def _zrow(b, _):
            for q in range(_CW // 16):
                rows_v[b, pl.ds(q * 16, 16)] = jnp.zeros((16,), jnp.float32)
            return 0
        lax.fori_loop(0, _B, _zrow, 0)
        for z in range(rslice // _B):
            pltpu.sync_copy(
                rows_v, accum.at[pl.ds(sid * rslice + z * _B, _B)])
        plsc.subcore_barrier()

        def _batch(j, _):
            base = sid * ept + j * _B
            pltpu.sync_copy(src_ref.at[pl.ds(base, _B)], src_v)
            pltpu.sync_copy(dst_ref.at[pl.ds(base, _B)], dst_v)
            pltpu.sync_copy(alpha_ref.at[pl.ds(base * H, _B * H)], alpha_v)
            cbase = c * N
            for q in range(_B // 16):
                idx_v[pl.ds(q * 16, 16)] = src_v[pl.ds(q * 16, 16)] + cbase
            pltpu.async_copy(feat_ref.at[idx_v], rows_v, sem).wait()

            epg = 16 // H
            def _scale(g, _):
                ev = alpha_v[pl.ds(16 * g, 16)]
                for r in range(epg):
                    b = epg * g + r
                    for s in range(seg):
                        h = (c * seg + s) if H > 1 else 0
                        m = _bcast(ev, r * H + h)
                        for q in range(scols // 16):
                            col = s * scols + q * 16
                            rows_v[b, pl.ds(col, 16)] = (
                                rows_v[b, pl.ds(col, 16)] * m)
                return 0
            lax.fori_loop(0, _B // epg, _scale, 0)

            pltpu.sync_copy(rows_v, accum.at[dst_v], add=True)
            return 0
        lax.fori_loop(0, nb, _batch, 0)
        plsc.subcore_barrier()

        pltpu.sync_copy(accum.at[pl.ds(sid * rslice, rslice)],
                        out_ref.at[c, pl.ds(sid * rslice, rslice)])
        plsc.subcore_barrier()


def _agg_sc(featperm, srcp, dstp, alphap, nchunks, H):
    ep = srcp.shape[0]
    ept = ep // _NT
    alphap = alphap.reshape(-1)
    mesh = plsc.VectorSubcoreMesh(core_axis_name="c", subcore_axis_name="s")
    f = pl.kernel(
        functools.partial(_agg_body, nchunks, H, ept),
        out_type=jax.ShapeDtypeStruct((nchunks, _NP, _CW), jnp.float32),
        mesh=mesh,
        scratch_types=[
            pltpu.VMEM((_B,), jnp.int32),
            pltpu.VMEM((_B,), jnp.int32),
            pltpu.VMEM((_B * H,), jnp.float32),
            pltpu.VMEM((_B,), jnp.int32),
            pltpu.VMEM((_B, _CW), jnp.float32),
            pltpu.VMEM_SHARED((_NP, _CW), jnp.float32),
            pltpu.SemaphoreType.DMA,
        ],
    )
    return f(featperm, srcp, dstp, alphap)


def _edge_alpha_xla(el, er, src, dst):
    e = jax.nn.leaky_relu(el[src] + er[dst], negative_slope=0.2)
    emax = jax.ops.segment_max(e, dst, num_segments=N)
    emax = jnp.where(jnp.isfinite(emax), emax, 0.0)
    ee = jnp.exp(e - emax[dst])
    denom = jax.ops.segment_sum(ee, dst, num_segments=N)
    return ee / jnp.maximum(denom[dst], 1e-9)


def _pad_edges(x, ep, fill):
    pad = jnp.full((ep - E,) + x.shape[1:], fill, x.dtype)
    return jnp.concatenate([x, pad], axis=0)


def kernel(node_2d_features, feat_2d, edge_index, W1, al1, ar1, b1, W2, al2,
           ar2, b2, Wc, bc):
    src, dst = edge_index[0], edge_index[1]
    ep = _NT * _B * -(-E // (_NT * _B))
    srcp = _pad_edges(src, ep, 0)
    dstp = _pad_edges(dst, ep, N)

    h0 = jnp.concatenate([node_2d_features, feat_2d], axis=1)
    eye = jnp.eye(H1, dtype=jnp.float32)
    A1l = (al1.reshape(H1, 1, HID) * eye[:, :, None]).transpose(1, 2, 0).reshape(H1 * HID, H1)
    A1r = (ar1.reshape(H1, 1, HID) * eye[:, :, None]).transpose(1, 2, 0).reshape(H1 * HID, H1)
    A1 = jnp.concatenate([A1l, A1r], axis=1)
    feat1, elr1 = _dense1(h0, W1, A1)
    el1, er1 = elr1[:, :H1], elr1[:, H1:]

    alpha1 = _pad_edges(_edge_alpha_xla(el1, er1, src, dst), ep, 0.0)
    featperm1 = feat1.reshape(N, 4, _CW).transpose(1, 0, 2).reshape(4 * N, _CW)
    agg1 = _agg_sc(featperm1, srcp, dstp, alpha1, 4, H1)
    rst1 = agg1[:, :N, :].transpose(1, 0, 2).reshape(N, H1 * HID)
    h1 = jax.nn.elu(rst1 + b1[None, :])

    A2 = jnp.concatenate([al2.reshape(OUT, 1), ar2.reshape(OUT, 1)], axis=1)
    feat2, elr2 = _dense1(h1, W2, A2)
    el2, er2 = elr2[:, :1], elr2[:, 1:]

    alpha2 = _pad_edges(_edge_alpha_xla(el2, er2, src, dst), ep, 0.0)
    featperm2 = feat2.reshape(N, 2, _CW).transpose(1, 0, 2).reshape(2 * N, _CW)
    agg2 = _agg_sc(featperm2, srcp, dstp, alpha2, 2, 1)
    rst2 = agg2[:, :N, :].transpose(1, 0, 2).reshape(N, OUT)
    h2 = rst2 + b2[None, :]

    y = h2.mean(axis=0, keepdims=True)
    label = y @ Wc + bc
    return (h2, label)
